# Initial kernel scaffold; baseline (speedup 1.0000x reference)
import jax
import jax.numpy as jnp
from jax.experimental import pallas as pl


def kernel(time, abundances, temperature, cr_rate, fuv_rate, incidence, alpha, beta, gamma, alpha_cr, alpha_fuv, species_idx):
    raise NotImplementedError("write your pallas kernel here")


# speedup vs baseline: 5.2310x; 5.2310x over previous
import functools

import jax
import jax.numpy as jnp
from jax import lax
from jax.experimental import pallas as pl
from jax.experimental.pallas import tpu as pltpu
from jax.experimental.pallas import tpu_sc as plsc

S = 1024
R = 65536

_NC = 2
_NS = 16
_LANES = 16
_NW = _NC * _NS
_CHUNK = R // _NW


def _sc_factor_body(ab_hbm, idx0_hbm, idx1_hbm, out_hbm, ab_v, i0_v, i1_v, f_v):
    wid = lax.axis_index("s") * _NC + lax.axis_index("c")
    base = wid * _CHUNK
    pltpu.sync_copy(ab_hbm, ab_v)
    pltpu.sync_copy(idx0_hbm.at[pl.ds(base, _CHUNK)], i0_v)
    pltpu.sync_copy(idx1_hbm.at[pl.ds(base, _CHUNK)], i1_v)

    def step(i, carry):
        off = i * _LANES
        iv0 = i0_v[pl.ds(off, _LANES)]
        iv1 = i1_v[pl.ds(off, _LANES)]
        a0 = plsc.load_gather(ab_v, [iv0])
        a1 = plsc.load_gather(ab_v, [iv1])
        f_v[pl.ds(off, _LANES)] = a0 * a1
        return carry

    lax.fori_loop(0, _CHUNK // _LANES, step, 0, unroll=4)
    pltpu.sync_copy(f_v, out_hbm.at[pl.ds(base, _CHUNK)])


_sc_factor = pl.kernel(
    _sc_factor_body,
    out_type=jax.ShapeDtypeStruct((R,), jnp.float32),
    mesh=plsc.VectorSubcoreMesh(
        core_axis_name="c", subcore_axis_name="s", num_cores=_NC,
        num_subcores=_NS),
    scratch_types=[
        pltpu.VMEM((S,), jnp.float32),
        pltpu.VMEM((_CHUNK,), jnp.int32),
        pltpu.VMEM((_CHUNK,), jnp.int32),
        pltpu.VMEM((_CHUNK,), jnp.float32),
    ],
    compiler_params=pltpu.CompilerParams(needs_layout_passes=False),
)


_RB = 2048
_KSTEPS = R // _RB


def _tc_matvec_body(s_ref, inc_ref, al_ref, be_ref, ga_ref, ac_ref, af_ref,
                    fa_ref, out_ref):
    lt = s_ref[0, 0]
    ninvT = s_ref[0, 1]
    cr = s_ref[0, 2]
    fuv = s_ref[0, 3]
    rates = (al_ref[...] * jnp.exp(be_ref[...] * lt + ga_ref[...] * ninvT)
             + ac_ref[...] * cr + af_ref[...] * fuv)
    v = rates * fa_ref[...]
    part = lax.dot_general(
        inc_ref[...], v, dimension_numbers=(((1,), (1,)), ((), ())),
        preferred_element_type=jnp.float32)

    @pl.when(pl.program_id(0) == 0)
    def _init():
        out_ref[...] = jnp.zeros_like(out_ref)

    out_ref[...] += part


def _vec_spec():
    return pl.BlockSpec((1, _RB), lambda i: (0, i))


_tc_matvec = pl.pallas_call(
    _tc_matvec_body,
    grid=(_KSTEPS,),
    in_specs=[
        pl.BlockSpec(memory_space=pltpu.SMEM),
        pl.BlockSpec((S, _RB), lambda i: (0, i)),
        _vec_spec(), _vec_spec(), _vec_spec(), _vec_spec(), _vec_spec(),
        _vec_spec(),
    ],
    out_specs=pl.BlockSpec((S, 1), lambda i: (0, 0)),
    out_shape=jax.ShapeDtypeStruct((S, 1), jnp.float32),
    compiler_params=pltpu.CompilerParams(
        dimension_semantics=("arbitrary",)),
)


def kernel(time, abundances, temperature, cr_rate, fuv_rate, incidence,
           alpha, beta, gamma, alpha_cr, alpha_fuv, species_idx):
    idx2 = species_idx.reshape(R, 2)
    idx0 = idx2[:, 0]
    idx1 = idx2[:, 1]

    factor = _sc_factor(abundances, idx0, idx1)

    scal = jnp.stack([
        jnp.log(temperature / 300.0),
        -1.0 / temperature,
        cr_rate,
        fuv_rate,
    ]).reshape(1, 4)

    out = _tc_matvec(
        scal, incidence,
        alpha.reshape(1, R), beta.reshape(1, R), gamma.reshape(1, R),
        alpha_cr.reshape(1, R), alpha_fuv.reshape(1, R),
        factor.reshape(1, R))
    return out.reshape(S)
